# Initial kernel scaffold; baseline (speedup 1.0000x reference)
import jax
import jax.numpy as jnp
from jax.experimental import pallas as pl


def kernel(x_sparse, x_dense, lin_tables, emb_tables, o2W, o2b, lin_dense_W, lin_dense_b, dW1, db1, dW2, db2, deepW1, deepb1, deepW2, deepb2, fcW, fcb):
    raise NotImplementedError("write your pallas kernel here")


# speedup vs baseline: 7.9419x; 7.9419x over previous
import functools

import jax
import jax.numpy as jnp
from jax import lax
from jax.experimental import pallas as pl
from jax.experimental.pallas import tpu as pltpu
from jax.experimental.pallas import tpu_sc as plsc

_F = 26
_V = 100000
_U = 32
_E = 16
_D = 13
_B = 16384
_H1 = 128
_H2 = 64
_DNN_W = 0.5

_NC, _NS = 2, 16
_NW = _NC * _NS
_ROWS_W = _B * _F // _NW
_CHUNK = 128
_KFIRE = 8
_GROUP = _CHUNK * _KFIRE
_NGROUP = _ROWS_W // _GROUP
_IDXROWS_W = _ROWS_W // _CHUNK


def _sc_gather_body(emb_hbm, lin_hbm, idx_hbm, g_hbm, linv_hbm,
                    idx_v, rows_v, lin_v, gsem, lsem):
    wid = lax.axis_index("s") * _NC + lax.axis_index("c")
    pltpu.sync_copy(idx_hbm.at[pl.ds(wid * _IDXROWS_W, _IDXROWS_W)], idx_v)
    out_base = wid * _ROWS_W

    def group(jj, carry):
        waits = []
        for i in range(_KFIRE):
            ix = idx_v.at[jj * _KFIRE + i]
            waits.append(pltpu.async_copy(
                emb_hbm.at[ix], rows_v.at[pl.ds(i * _CHUNK, _CHUNK)], gsem))
            waits.append(pltpu.async_copy(
                lin_hbm.at[ix], lin_v.at[pl.ds(i * _CHUNK, _CHUNK)], lsem))
        for w in waits:
            w.wait()
        off = out_base + jj * _GROUP
        pltpu.sync_copy(rows_v, g_hbm.at[pl.ds(off, _GROUP)])
        pltpu.sync_copy(lin_v, linv_hbm.at[pl.ds(off, _GROUP)])
        return carry

    lax.fori_loop(0, _NGROUP, group, 0)


_sc_gather = functools.partial(
    pl.kernel,
    out_type=[jax.ShapeDtypeStruct((_B * _F, _U), jnp.float32),
              jax.ShapeDtypeStruct((_B * _F,), jnp.float32)],
    mesh=plsc.VectorSubcoreMesh(core_axis_name="c", subcore_axis_name="s"),
    scratch_types=[pltpu.VMEM((_IDXROWS_W, _CHUNK), jnp.int32),
                   pltpu.VMEM((_GROUP, _U), jnp.float32),
                   pltpu.VMEM((_GROUP,), jnp.float32),
                   pltpu.SemaphoreType.DMA,
                   pltpu.SemaphoreType.DMA],
    compiler_params=pltpu.CompilerParams(use_tc_tiling_on_sc=False),
)(_sc_gather_body)


_BB = 512


def _tc_body(g_ref, linv_ref, xd_ref, o2wbd_ref, o2bf_ref, ssel_ref,
             dw1f_ref, db1f_ref, dw2bd_ref, db2f_ref, w1_ref, b1_ref,
             w2_ref, b2_ref, fcw_ref, fcb_ref, ldw_ref, ldb_ref,
             gsum_ref, o_ref):
    f32 = jnp.float32
    g = jnp.maximum(g_ref[...], 0.0)
    o2s = jnp.dot(g, o2wbd_ref[...], preferred_element_type=f32) + o2bf_ref[...]
    xd = xd_ref[...]
    xrep = jnp.dot(xd, ssel_ref[...], preferred_element_type=f32)
    t = jnp.maximum(xrep * dw1f_ref[...] + db1f_ref[...], 0.0)
    o2d = jnp.dot(t, dw2bd_ref[...], preferred_element_type=f32) + db2f_ref[...]
    order2 = jnp.concatenate([o2s, o2d], axis=1)
    deep = jnp.maximum(jnp.dot(order2, w1_ref[...], preferred_element_type=f32)
                       + b1_ref[...], 0.0)
    deep = jnp.maximum(jnp.dot(deep, w2_ref[...], preferred_element_type=f32)
                       + b2_ref[...], 0.0)
    deep = jnp.dot(deep, fcw_ref[...], preferred_element_type=f32) + fcb_ref[...]
    sum_vec = jnp.dot(order2, gsum_ref[...], preferred_element_type=f32)
    sq_vec = jnp.dot(order2 * order2, gsum_ref[...], preferred_element_type=f32)
    cross = 0.5 * jnp.sum(sum_vec * sum_vec - sq_vec, axis=1, keepdims=True)
    linear = (jnp.sum(linv_ref[...], axis=1, keepdims=True)
              + jnp.dot(xd, ldw_ref[...], preferred_element_type=f32)
              + ldb_ref[...])
    o_ref[...] = linear + cross + _DNN_W * deep


def _tc_forward(g, linv, xd, o2wbd, o2bf, ssel, dw1f, db1f, dw2bd, db2f,
                w1, b1, w2, b2, fcw, fcb, ldw, ldb, gsum):
    nblk = _B // _BB
    row_spec = lambda a: pl.BlockSpec((_BB, a.shape[1]), lambda i: (i, 0))
    full_spec = lambda a: pl.BlockSpec(a.shape, lambda i: (0, 0))
    in_specs = [row_spec(g), row_spec(linv), row_spec(xd)] + [
        full_spec(a) for a in (o2wbd, o2bf, ssel, dw1f, db1f, dw2bd, db2f,
                               w1, b1, w2, b2, fcw, fcb, ldw, ldb, gsum)]
    return pl.pallas_call(
        _tc_body,
        grid=(nblk,),
        in_specs=in_specs,
        out_specs=pl.BlockSpec((_BB, 1), lambda i: (i, 0)),
        out_shape=jax.ShapeDtypeStruct((_B, 1), jnp.float32),
        compiler_params=pltpu.CompilerParams(
            dimension_semantics=("arbitrary",)),
    )(g, linv, xd, o2wbd, o2bf, ssel, dw1f, db1f, dw2bd, db2f,
      w1, b1, w2, b2, fcw, fcb, ldw, ldb, gsum)


def kernel(x_sparse, x_dense, lin_tables, emb_tables, o2W, o2b,
           lin_dense_W, lin_dense_b, dW1, db1, dW2, db2,
           deepW1, deepb1, deepW2, deepb2, fcW, fcb):
    f32 = jnp.float32
    flat_idx = (x_sparse.astype(jnp.int32)
                + (jnp.arange(_F, dtype=jnp.int32) * _V)[None, :]).reshape(-1)
    idx2d = flat_idx.reshape(_B * _F // _CHUNK, _CHUNK)

    g_flat, lin_flat = _sc_gather(
        emb_tables.reshape(_F * _V, _U), lin_tables.reshape(_F * _V), idx2d)

    eyeF = jnp.eye(_F, dtype=f32)
    o2wbd = (eyeF[:, None, :, None] * o2W[:, :, None, :]).reshape(_F * _U, _F * _E)
    eyeD = jnp.eye(_D, dtype=f32)
    dw2bd = (eyeD[:, None, :, None] * dW2[:, :, None, :]).reshape(_D * _U, _D * _E)
    ssel = jnp.repeat(eyeD, _U, axis=1)
    gsum = jnp.tile(jnp.eye(_E, dtype=f32), (_F + _D, 1))

    out = _tc_forward(
        g_flat.reshape(_B, _F * _U), lin_flat.reshape(_B, _F), x_dense,
        o2wbd, o2b.reshape(1, _F * _E), ssel,
        dW1.reshape(1, _D * _U), db1.reshape(1, _D * _U),
        dw2bd, db2.reshape(1, _D * _E),
        deepW1, deepb1.reshape(1, _H1), deepW2, deepb2.reshape(1, _H2),
        fcW, fcb.reshape(1, 1), lin_dense_W, lin_dense_b.reshape(1, 1), gsum)
    return out
